# Initial kernel scaffold; baseline (speedup 1.0000x reference)
import jax
import jax.numpy as jnp
from jax.experimental import pallas as pl


def kernel(x0, x1, x2, ei0, ei1, ego_W, ego_b, W0, b0, W1, b1):
    raise NotImplementedError("write your pallas kernel here")


# speedup vs baseline: 16.4889x; 16.4889x over previous
import functools

import jax
import jax.numpy as jnp
from jax import lax
from jax.experimental import pallas as pl
from jax.experimental.pallas import tpu as pltpu
from jax.experimental.pallas import tpu_sc as plsc

N = 10000
D = 128
E = 320000
NC = 2
NS = 16
L = 16
NW = NC * NS
N_PAD = 10240
R_PER_S = N_PAD // NS
E_PER_W = E // NW
CHUNK = 128
N_FULL = E_PER_W // CHUNK
TAIL = E_PER_W - N_FULL * CHUNK

_mesh = plsc.VectorSubcoreMesh(core_axis_name="c", subcore_axis_name="s")


def _wid():
    return lax.axis_index("s") * NC + lax.axis_index("c")


E_PER_S = E // NS
D_FULL = E_PER_S // CHUNK
D_TAIL = E_PER_S - D_FULL * CHUNK
R_DEG = N_PAD // NS


def _deg_body(dst0_hbm, dst1_hbm, ones_hbm, zeros_hbm, out0_hbm, out1_hbm,
              idx_v, idxt_v, ones_v, sh):
    c = lax.axis_index("c")
    s = lax.axis_index("s")
    pltpu.sync_copy(zeros_hbm, sh.at[pl.ds(s * R_DEG, R_DEG)])
    pltpu.sync_copy(ones_hbm, ones_v)
    plsc.subcore_barrier()

    base = s * E_PER_S

    def count(dst_hbm):
        @pl.loop(0, D_FULL)
        def _(k):
            pltpu.sync_copy(dst_hbm.at[pl.ds(base + k * CHUNK, CHUNK)], idx_v)
            pltpu.sync_copy(ones_v, sh.at[idx_v], add=True)
        pltpu.sync_copy(dst_hbm.at[pl.ds(base + D_FULL * CHUNK, D_TAIL)], idxt_v)
        pltpu.sync_copy(ones_v.at[pl.ds(0, D_TAIL)], sh.at[idxt_v], add=True)

    @pl.when(c == 0)
    def _():
        count(dst0_hbm)

    @pl.when(c == 1)
    def _():
        count(dst1_hbm)

    plsc.subcore_barrier()
    sl = pl.ds(s * R_DEG, R_DEG)

    @pl.when(c == 0)
    def _():
        pltpu.sync_copy(sh.at[sl], out0_hbm.at[sl])

    @pl.when(c == 1)
    def _():
        pltpu.sync_copy(sh.at[sl], out1_hbm.at[sl])


def _deg_call(dst0, dst1, onesD, zerosDeg):
    out = jax.ShapeDtypeStruct((N_PAD, D), jnp.float32)
    k = pl.kernel(
        _deg_body,
        out_type=(out, out),
        mesh=_mesh,
        scratch_types=[
            pltpu.VMEM((CHUNK,), jnp.int32),
            pltpu.VMEM((D_TAIL,), jnp.int32),
            pltpu.VMEM((CHUNK, D), jnp.float32),
            pltpu.VMEM_SHARED((N_PAD, D), jnp.float32),
        ],
    )
    return k(dst0, dst1, onesD, zerosDeg)


def _pay_body(y_hbm, src_hbm, dst_hbm, zeros_hbm, out_hbm,
              src_v, dst_v, srct_v, dstt_v, rows_v, rowst_v, sh, sem):
    c = lax.axis_index("c")
    s = lax.axis_index("s")
    wid = _wid()
    pltpu.sync_copy(zeros_hbm, sh.at[pl.ds(s * R_PER_S, R_PER_S)])
    plsc.subcore_barrier()

    base = wid * E_PER_W

    @pl.loop(0, N_FULL)
    def _(k):
        pltpu.sync_copy(src_hbm.at[pl.ds(base + k * CHUNK, CHUNK)], src_v)
        pltpu.sync_copy(dst_hbm.at[pl.ds(base + k * CHUNK, CHUNK)], dst_v)
        pltpu.async_copy(y_hbm.at[src_v], rows_v, sem).wait()
        pltpu.sync_copy(rows_v, sh.at[dst_v], add=True)

    pltpu.sync_copy(src_hbm.at[pl.ds(base + N_FULL * CHUNK, TAIL)], srct_v)
    pltpu.sync_copy(dst_hbm.at[pl.ds(base + N_FULL * CHUNK, TAIL)], dstt_v)
    pltpu.async_copy(y_hbm.at[srct_v], rowst_v, sem).wait()
    pltpu.sync_copy(rowst_v, sh.at[dstt_v], add=True)

    plsc.subcore_barrier()
    row = c * N_PAD + s * R_PER_S
    pltpu.sync_copy(sh.at[pl.ds(s * R_PER_S, R_PER_S)], out_hbm.at[pl.ds(row, R_PER_S)])


def _pay_call(y, src, dst, zerosD):
    k = pl.kernel(
        _pay_body,
        out_type=jax.ShapeDtypeStruct((NC * N_PAD, D), jnp.float32),
        mesh=_mesh,
        scratch_types=[
            pltpu.VMEM((CHUNK,), jnp.int32),
            pltpu.VMEM((CHUNK,), jnp.int32),
            pltpu.VMEM((TAIL,), jnp.int32),
            pltpu.VMEM((TAIL,), jnp.int32),
            pltpu.VMEM((CHUNK, D), jnp.float32),
            pltpu.VMEM((TAIL, D), jnp.float32),
            pltpu.VMEM_SHARED((N_PAD, D), jnp.float32),
            pltpu.SemaphoreType.DMA,
        ],
    )
    return k(y, src, dst, zerosD)


BLK = 2000


def _prep_body(x0_ref, x1_ref, x2_ref, egoW_ref, egob_ref, W0_ref, W1_ref,
               d0_ref, d1_ref, hidden_ref, y0_ref, y1_ref):
    hidden_ref[...] = (
        jnp.dot(x0_ref[...], egoW_ref[...], preferred_element_type=jnp.float32)
        + egob_ref[...]
    )
    dinv0 = lax.rsqrt(d0_ref[:, 0:1] + 1.0)
    y0_ref[...] = dinv0 * jnp.dot(x1_ref[...], W0_ref[...],
                                  preferred_element_type=jnp.float32)
    dinv1 = lax.rsqrt(d1_ref[:, 0:1] + 1.0)
    y1_ref[...] = dinv1 * jnp.dot(x2_ref[...], W1_ref[...],
                                  preferred_element_type=jnp.float32)


def _final_body(hidden_ref, y0_ref, y1_ref, a0a_ref, a0b_ref, a1a_ref, a1b_ref,
                d0_ref, d1_ref, b0_ref, b1_ref, out_ref):
    dinv0 = lax.rsqrt(d0_ref[:, 0:1] + 1.0)
    p0 = dinv0 * (a0a_ref[...] + a0b_ref[...] + y0_ref[...]) + b0_ref[...]
    dinv1 = lax.rsqrt(d1_ref[:, 0:1] + 1.0)
    p1 = dinv1 * (a1a_ref[...] + a1b_ref[...] + y1_ref[...]) + b1_ref[...]
    out_ref[...] = (hidden_ref[...] + jnp.maximum(p0, 0.0) + jnp.maximum(p1, 0.0))


def _row_spec(w):
    return pl.BlockSpec((BLK, w), lambda i: (i, 0))


def _full_spec(h, w):
    return pl.BlockSpec((h, w), lambda i: (0, 0))


def _prep_call(x0, x1, x2, egoW, egob, W0, W1, d0, d1):
    outs = [jax.ShapeDtypeStruct((N, D), jnp.float32)] * 3
    return pl.pallas_call(
        _prep_body,
        grid=(N // BLK,),
        in_specs=[
            _row_spec(D), _row_spec(D), _row_spec(D),
            _full_spec(D, D), _full_spec(1, D), _full_spec(D, D), _full_spec(D, D),
            _row_spec(D), _row_spec(D),
        ],
        out_specs=[_row_spec(D)] * 3,
        out_shape=outs,
    )(x0, x1, x2, egoW, egob, W0, W1, d0, d1)


def _final_call(hidden, y0, y1, a0a, a0b, a1a, a1b, d0, d1, b0, b1):
    return pl.pallas_call(
        _final_body,
        grid=(N // BLK,),
        in_specs=[
            _row_spec(D), _row_spec(D), _row_spec(D),
            _row_spec(D), _row_spec(D), _row_spec(D), _row_spec(D),
            _row_spec(D), _row_spec(D),
            _full_spec(1, D), _full_spec(1, D),
        ],
        out_specs=_row_spec(D),
        out_shape=jax.ShapeDtypeStruct((N, D), jnp.float32),
    )(hidden, y0, y1, a0a, a0b, a1a, a1b, d0, d1, b0, b1)


def kernel(x0, x1, x2, ei0, ei1, ego_W, ego_b, W0, b0, W1, b1):
    src0, dst0 = ei0[0], ei0[1]
    src1, dst1 = ei1[0], ei1[1]
    onesD = jnp.ones((CHUNK, D), jnp.float32)
    zerosDeg = jnp.zeros((R_DEG, D), jnp.float32)
    zerosD = jnp.zeros((R_PER_S, D), jnp.float32)

    deg0, deg1 = _deg_call(dst0, dst1, onesD, zerosDeg)
    d0, d1 = deg0[:N], deg1[:N]

    hidden, y0, y1 = _prep_call(x0, x1, x2, ego_W, ego_b.reshape(1, D), W0, W1,
                                d0, d1)

    acc0 = _pay_call(y0, src0, dst0, zerosD)
    acc1 = _pay_call(y1, src1, dst1, zerosD)
    a0a, a0b = acc0[:N], acc0[N_PAD:N_PAD + N]
    a1a, a1b = acc1[:N], acc1[N_PAD:N_PAD + N]

    return _final_call(hidden, y0, y1, a0a, a0b, a1a, a1b, d0, d1,
                       b0.reshape(1, D), b1.reshape(1, D))


# speedup vs baseline: 29.1078x; 1.7653x over previous
import functools

import jax
import jax.numpy as jnp
from jax import lax
from jax.experimental import pallas as pl
from jax.experimental.pallas import tpu as pltpu
from jax.experimental.pallas import tpu_sc as plsc

N = 10000
D = 128
E = 320000
NC = 2
NS = 16
NW = NC * NS
N_PAD = 10240
R_SUB = N_PAD // NS
CHUNK = 100
E_PER_S = E // NS
NCH = E_PER_S // CHUNK
MAIN = (NCH // 4) * 4 - 4

_mesh = plsc.VectorSubcoreMesh(core_axis_name="c", subcore_axis_name="s")


DEG_GRP = 10


def _deg_body(dst0_hbm, dst1_hbm, ones_hbm, zeros_hbm, out0_hbm, out1_hbm,
              idx_v, ones_v, sh, sem):
    c = lax.axis_index("c")
    s = lax.axis_index("s")
    pltpu.sync_copy(zeros_hbm, sh.at[pl.ds(s * R_SUB, R_SUB)])
    pltpu.sync_copy(ones_hbm, ones_v)

    def count(dstR_hbm):
        pltpu.sync_copy(dstR_hbm.at[pl.ds(s * NCH, NCH)], idx_v)
        plsc.subcore_barrier()

        @pl.loop(0, NCH // DEG_GRP)
        def _(g):
            for j in range(DEG_GRP):
                pltpu.async_copy(ones_v, sh.at[idx_v.at[g * DEG_GRP + j]],
                                 sem, add=True)
            for j in range(DEG_GRP):
                pltpu.make_async_copy(ones_v, sh.at[idx_v.at[g * DEG_GRP + j]],
                                      sem).wait()

    @pl.when(c == 0)
    def _():
        count(dst0_hbm)

    @pl.when(c == 1)
    def _():
        count(dst1_hbm)

    plsc.subcore_barrier()
    sl = pl.ds(s * R_SUB, R_SUB)

    @pl.when(c == 0)
    def _():
        pltpu.sync_copy(sh.at[sl], out0_hbm.at[sl])

    @pl.when(c == 1)
    def _():
        pltpu.sync_copy(sh.at[sl], out1_hbm.at[sl])


def _deg_call(dst0R, dst1R, onesD, zerosD):
    out = jax.ShapeDtypeStruct((N_PAD, D), jnp.float32)
    k = pl.kernel(
        _deg_body,
        out_type=(out, out),
        mesh=_mesh,
        scratch_types=[
            pltpu.VMEM((NCH, CHUNK), jnp.int32),
            pltpu.VMEM((CHUNK, D), jnp.float32),
            pltpu.VMEM_SHARED((N_PAD, D), jnp.float32),
            pltpu.SemaphoreType.DMA,
        ],
    )
    return k(dst0R, dst1R, onesD, zerosD)


SG = 8
NSTAGE = NCH // SG
NPAIR = NCH // 2


def _pay_body(y0_hbm, y1_hbm, src0R, dst0R, src1R, dst1R, zeros_hbm,
              out0_hbm, out1_hbm,
              srcA, dstA, srcB, dstB, rows0, rows1, sh,
              gsem0, gsem1, ssem0, ssem1, isem):
    c = lax.axis_index("c")
    s = lax.axis_index("s")

    def pipeline(y_hbm, srcR_hbm, dstR_hbm, out_hbm):
        base = s * NCH
        pltpu.sync_copy(zeros_hbm, sh.at[pl.ds(s * R_SUB, R_SUB)])
        pltpu.sync_copy(srcR_hbm.at[pl.ds(base, SG)], srcA)
        pltpu.sync_copy(dstR_hbm.at[pl.ds(base, SG)], dstA)
        pltpu.async_copy(srcR_hbm.at[pl.ds(base + SG, SG)], srcB, isem)
        pltpu.async_copy(dstR_hbm.at[pl.ds(base + SG, SG)], dstB, isem)
        plsc.subcore_barrier()

        def g_start(srcI, r, rows, gsem):
            pltpu.async_copy(y_hbm.at[srcI.at[r]], rows, gsem)

        def g_wait(srcI, r, rows, gsem):
            pltpu.make_async_copy(y_hbm.at[srcI.at[r]], rows, gsem).wait()

        def s_fire(dstI, r, rows, ssem):
            pltpu.async_copy(rows, sh.at[dstI.at[r]], ssem, add=True)

        def s_drain(dstI, r, rows, ssem):
            pltpu.make_async_copy(rows, sh.at[dstI.at[r]], ssem).wait()

        def i_start(srcI, dstI, t):
            pltpu.async_copy(srcR_hbm.at[pl.ds(base + t * SG, SG)], srcI, isem)
            pltpu.async_copy(dstR_hbm.at[pl.ds(base + t * SG, SG)], dstI, isem)

        def i_wait(srcI, dstI, t):
            pltpu.make_async_copy(srcR_hbm.at[pl.ds(base + t * SG, SG)],
                                  srcI, isem).wait()
            pltpu.make_async_copy(dstR_hbm.at[pl.ds(base + t * SG, SG)],
                                  dstI, isem).wait()

        g_start(srcA, 0, rows0, gsem0)

        def pair_body(p, srcC, dstC, srcN, dstN):
            q = p % 4
            r0 = q * 2
            r1 = q * 2 + 1
            t = p // 4

            @pl.when(jnp.logical_and(p >= 1, q >= 1))
            def _():
                s_drain(dstC, r1 - 2, rows1, ssem1)

            @pl.when(jnp.logical_and(p >= 1, q == 0))
            def _():
                s_drain(dstN, SG - 1, rows1, ssem1)

            @pl.when(jnp.logical_and(q == 0, t + 1 <= NSTAGE - 1))
            def _():
                @pl.when(p >= 4)
                def _():
                    i_start(srcN, dstN, t + 1)

            g_start(srcC, r1, rows1, gsem1)
            g_wait(srcC, r0, rows0, gsem0)
            s_fire(dstC, r0, rows0, ssem0)
            s_drain(dstC, r0, rows0, ssem0)

            @pl.when(jnp.logical_and(q < 3, p < NPAIR - 1))
            def _():
                g_start(srcC, r0 + 2, rows0, gsem0)

            @pl.when(jnp.logical_and(q == 3, p < NPAIR - 1))
            def _():
                i_wait(srcN, dstN, t + 1)
                g_start(srcN, 0, rows0, gsem0)

            g_wait(srcC, r1, rows1, gsem1)
            s_fire(dstC, r1, rows1, ssem1)

        @pl.loop(0, NPAIR)
        def _(p):
            slot = (p // 4) % 2

            @pl.when(slot == 0)
            def _():
                pair_body(p, srcA, dstA, srcB, dstB)

            @pl.when(slot == 1)
            def _():
                pair_body(p, srcB, dstB, srcA, dstA)

        s_drain(dstA, SG - 1, rows1, ssem1)

        plsc.subcore_barrier()
        sl = pl.ds(s * R_SUB, R_SUB)
        pltpu.sync_copy(sh.at[sl], out_hbm.at[sl])

    @pl.when(c == 0)
    def _():
        pipeline(y0_hbm, src0R, dst0R, out0_hbm)

    @pl.when(c == 1)
    def _():
        pipeline(y1_hbm, src1R, dst1R, out1_hbm)


def _pay_call(y0, y1, src0R, dst0R, src1R, dst1R, zerosD):
    out = jax.ShapeDtypeStruct((N_PAD, D), jnp.float32)
    k = pl.kernel(
        _pay_body,
        out_type=(out, out),
        mesh=_mesh,
        scratch_types=[
            pltpu.VMEM((SG, CHUNK), jnp.int32),
            pltpu.VMEM((SG, CHUNK), jnp.int32),
            pltpu.VMEM((SG, CHUNK), jnp.int32),
            pltpu.VMEM((SG, CHUNK), jnp.int32),
            pltpu.VMEM((CHUNK, D), jnp.float32),
            pltpu.VMEM((CHUNK, D), jnp.float32),
            pltpu.VMEM_SHARED((N_PAD, D), jnp.float32),
            pltpu.SemaphoreType.DMA,
            pltpu.SemaphoreType.DMA,
            pltpu.SemaphoreType.DMA,
            pltpu.SemaphoreType.DMA,
            pltpu.SemaphoreType.DMA,
        ],
    )
    return k(y0, y1, src0R, dst0R, src1R, dst1R, zerosD)


BLK = 2000


def _prep_body(x0_ref, x1_ref, x2_ref, egoW_ref, egob_ref, W0_ref, W1_ref,
               d0_ref, d1_ref, hidden_ref, y0_ref, y1_ref):
    hidden_ref[...] = (
        jnp.dot(x0_ref[...], egoW_ref[...], preferred_element_type=jnp.float32)
        + egob_ref[...]
    )
    dinv0 = lax.rsqrt(d0_ref[:, 0:1] + 1.0)
    y0_ref[...] = dinv0 * jnp.dot(x1_ref[...], W0_ref[...],
                                  preferred_element_type=jnp.float32)
    dinv1 = lax.rsqrt(d1_ref[:, 0:1] + 1.0)
    y1_ref[...] = dinv1 * jnp.dot(x2_ref[...], W1_ref[...],
                                  preferred_element_type=jnp.float32)


def _final_body(hidden_ref, y0_ref, y1_ref, a0_ref, a1_ref,
                d0_ref, d1_ref, b0_ref, b1_ref, out_ref):
    dinv0 = lax.rsqrt(d0_ref[:, 0:1] + 1.0)
    p0 = dinv0 * (a0_ref[...] + y0_ref[...]) + b0_ref[...]
    dinv1 = lax.rsqrt(d1_ref[:, 0:1] + 1.0)
    p1 = dinv1 * (a1_ref[...] + y1_ref[...]) + b1_ref[...]
    out_ref[...] = (hidden_ref[...] + jnp.maximum(p0, 0.0) + jnp.maximum(p1, 0.0))


def _row_spec(w):
    return pl.BlockSpec((BLK, w), lambda i: (i, 0))


def _full_spec(h, w):
    return pl.BlockSpec((h, w), lambda i: (0, 0))


def _prep_call(x0, x1, x2, egoW, egob, W0, W1, d0, d1):
    outs = [jax.ShapeDtypeStruct((N, D), jnp.float32)] * 3
    return pl.pallas_call(
        _prep_body,
        grid=(N // BLK,),
        in_specs=[
            _row_spec(D), _row_spec(D), _row_spec(D),
            _full_spec(D, D), _full_spec(1, D), _full_spec(D, D), _full_spec(D, D),
            _row_spec(D), _row_spec(D),
        ],
        out_specs=[_row_spec(D)] * 3,
        out_shape=outs,
    )(x0, x1, x2, egoW, egob, W0, W1, d0, d1)


def _final_call(hidden, y0, y1, a0, a1, d0, d1, b0, b1):
    return pl.pallas_call(
        _final_body,
        grid=(N // BLK,),
        in_specs=[
            _row_spec(D), _row_spec(D), _row_spec(D),
            _row_spec(D), _row_spec(D),
            _row_spec(D), _row_spec(D),
            _full_spec(1, D), _full_spec(1, D),
        ],
        out_specs=_row_spec(D),
        out_shape=jax.ShapeDtypeStruct((N, D), jnp.float32),
    )(hidden, y0, y1, a0, a1, d0, d1, b0, b1)


def kernel(x0, x1, x2, ei0, ei1, ego_W, ego_b, W0, b0, W1, b1):
    src0R = ei0[0].reshape(E // CHUNK, CHUNK)
    dst0R = ei0[1].reshape(E // CHUNK, CHUNK)
    src1R = ei1[0].reshape(E // CHUNK, CHUNK)
    dst1R = ei1[1].reshape(E // CHUNK, CHUNK)
    onesD = jnp.ones((CHUNK, D), jnp.float32)
    zerosD = jnp.zeros((R_SUB, D), jnp.float32)

    deg0, deg1 = _deg_call(dst0R, dst1R, onesD, zerosD)
    d0, d1 = deg0[:N], deg1[:N]

    hidden, y0, y1 = _prep_call(x0, x1, x2, ego_W, ego_b.reshape(1, D), W0, W1,
                                d0, d1)

    acc0, acc1 = _pay_call(y0, y1, src0R, dst0R, src1R, dst1R, zerosD)
    a0, a1 = acc0[:N], acc1[:N]

    return _final_call(hidden, y0, y1, a0, a1, d0, d1,
                       b0.reshape(1, D), b1.reshape(1, D))


# speedup vs baseline: 29.5028x; 1.0136x over previous
import functools

import jax
import jax.numpy as jnp
from jax import lax
from jax.experimental import pallas as pl
from jax.experimental.pallas import tpu as pltpu
from jax.experimental.pallas import tpu_sc as plsc

N = 10000
D = 128
E = 320000
NC = 2
NS = 16
NW = NC * NS
N_PAD = 10240
R_SUB = N_PAD // NS
CHUNK = 100
E_PER_S = E // NS
NCH = E_PER_S // CHUNK
MAIN = (NCH // 4) * 4 - 4

_mesh = plsc.VectorSubcoreMesh(core_axis_name="c", subcore_axis_name="s")


DEG_GRP = 10


def _deg_body(dst0_hbm, dst1_hbm, ones_hbm, zeros_hbm, out0_hbm, out1_hbm,
              idx_v, ones_v, sh, sem):
    c = lax.axis_index("c")
    s = lax.axis_index("s")
    pltpu.sync_copy(zeros_hbm, sh.at[pl.ds(s * R_SUB, R_SUB)])
    pltpu.sync_copy(ones_hbm, ones_v)

    def count(dstR_hbm):
        pltpu.sync_copy(dstR_hbm.at[pl.ds(s * NCH, NCH)], idx_v)
        plsc.subcore_barrier()

        @pl.loop(0, NCH // DEG_GRP)
        def _(g):
            for j in range(DEG_GRP):
                pltpu.async_copy(ones_v, sh.at[idx_v.at[g * DEG_GRP + j]],
                                 sem, add=True)
            for j in range(DEG_GRP):
                pltpu.make_async_copy(ones_v, sh.at[idx_v.at[g * DEG_GRP + j]],
                                      sem).wait()

    @pl.when(c == 0)
    def _():
        count(dst0_hbm)

    @pl.when(c == 1)
    def _():
        count(dst1_hbm)

    plsc.subcore_barrier()
    sl = pl.ds(s * R_SUB, R_SUB)

    @pl.when(c == 0)
    def _():
        pltpu.sync_copy(sh.at[sl], out0_hbm.at[sl])

    @pl.when(c == 1)
    def _():
        pltpu.sync_copy(sh.at[sl], out1_hbm.at[sl])


def _deg_call(dst0R, dst1R, onesD, zerosD):
    out = jax.ShapeDtypeStruct((N_PAD, D), jnp.float32)
    k = pl.kernel(
        _deg_body,
        out_type=(out, out),
        mesh=_mesh,
        scratch_types=[
            pltpu.VMEM((NCH, CHUNK), jnp.int32),
            pltpu.VMEM((CHUNK, D), jnp.float32),
            pltpu.VMEM_SHARED((N_PAD, D), jnp.float32),
            pltpu.SemaphoreType.DMA,
        ],
    )
    return k(dst0R, dst1R, onesD, zerosD)


SG = 8
NSTAGE = NCH // SG
NPAIR = NCH // 2


def _pay_body(y0_hbm, y1_hbm, src0R, dst0R, src1R, dst1R, zeros_hbm,
              out0_hbm, out1_hbm,
              srcA, dstA, srcB, dstB, rows0, rows1, sh,
              gsem0, gsem1, ssem0, ssem1, isem):
    c = lax.axis_index("c")
    s = lax.axis_index("s")

    def pipeline(y_hbm, srcR_hbm, dstR_hbm, out_hbm):
        base = s * NCH
        pltpu.sync_copy(zeros_hbm, sh.at[pl.ds(s * R_SUB, R_SUB)])
        pltpu.sync_copy(srcR_hbm.at[pl.ds(base, SG)], srcA)
        pltpu.sync_copy(dstR_hbm.at[pl.ds(base, SG)], dstA)
        pltpu.async_copy(srcR_hbm.at[pl.ds(base + SG, SG)], srcB, isem)
        pltpu.async_copy(dstR_hbm.at[pl.ds(base + SG, SG)], dstB, isem)
        plsc.subcore_barrier()

        def g_start(srcI, r, rows, gsem):
            pltpu.async_copy(y_hbm.at[srcI.at[r]], rows, gsem)

        def g_wait(srcI, r, rows, gsem):
            pltpu.make_async_copy(y_hbm.at[srcI.at[r]], rows, gsem).wait()

        def s_fire(dstI, r, rows, ssem):
            pltpu.async_copy(rows, sh.at[dstI.at[r]], ssem, add=True)

        def s_drain(dstI, r, rows, ssem):
            pltpu.make_async_copy(rows, sh.at[dstI.at[r]], ssem).wait()

        def i_start(srcI, dstI, t):
            pltpu.async_copy(srcR_hbm.at[pl.ds(base + t * SG, SG)], srcI, isem)
            pltpu.async_copy(dstR_hbm.at[pl.ds(base + t * SG, SG)], dstI, isem)

        def i_wait(srcI, dstI, t):
            pltpu.make_async_copy(srcR_hbm.at[pl.ds(base + t * SG, SG)],
                                  srcI, isem).wait()
            pltpu.make_async_copy(dstR_hbm.at[pl.ds(base + t * SG, SG)],
                                  dstI, isem).wait()

        g_start(srcA, 0, rows0, gsem0)

        def pair_body(p, srcC, dstC, srcN, dstN):
            q = p % 4
            r0 = q * 2
            r1 = q * 2 + 1
            t = p // 4

            @pl.when(jnp.logical_and(p >= 1, q >= 1))
            def _():
                s_drain(dstC, r1 - 2, rows1, ssem1)

            @pl.when(jnp.logical_and(p >= 1, q == 0))
            def _():
                s_drain(dstN, SG - 1, rows1, ssem1)

            @pl.when(jnp.logical_and(q == 0, t + 1 <= NSTAGE - 1))
            def _():
                @pl.when(p >= 4)
                def _():
                    i_start(srcN, dstN, t + 1)

            g_start(srcC, r1, rows1, gsem1)
            g_wait(srcC, r0, rows0, gsem0)
            s_fire(dstC, r0, rows0, ssem0)
            s_drain(dstC, r0, rows0, ssem0)

            @pl.when(jnp.logical_and(q < 3, p < NPAIR - 1))
            def _():
                g_start(srcC, r0 + 2, rows0, gsem0)

            @pl.when(jnp.logical_and(q == 3, p < NPAIR - 1))
            def _():
                i_wait(srcN, dstN, t + 1)
                g_start(srcN, 0, rows0, gsem0)

            g_wait(srcC, r1, rows1, gsem1)
            s_fire(dstC, r1, rows1, ssem1)

        @pl.loop(0, NPAIR)
        def _(p):
            slot = (p // 4) % 2

            @pl.when(slot == 0)
            def _():
                pair_body(p, srcA, dstA, srcB, dstB)

            @pl.when(slot == 1)
            def _():
                pair_body(p, srcB, dstB, srcA, dstA)

        s_drain(dstA, SG - 1, rows1, ssem1)

        plsc.subcore_barrier()
        sl = pl.ds(s * R_SUB, R_SUB)
        pltpu.sync_copy(sh.at[sl], out_hbm.at[sl])

    @pl.when(c == 0)
    def _():
        pipeline(y0_hbm, src0R, dst0R, out0_hbm)

    @pl.when(c == 1)
    def _():
        pipeline(y1_hbm, src1R, dst1R, out1_hbm)


def _pay_call(y0, y1, src0R, dst0R, src1R, dst1R, zerosD):
    out = jax.ShapeDtypeStruct((N_PAD, D), jnp.float32)
    k = pl.kernel(
        _pay_body,
        out_type=(out, out),
        mesh=_mesh,
        scratch_types=[
            pltpu.VMEM((SG, CHUNK), jnp.int32),
            pltpu.VMEM((SG, CHUNK), jnp.int32),
            pltpu.VMEM((SG, CHUNK), jnp.int32),
            pltpu.VMEM((SG, CHUNK), jnp.int32),
            pltpu.VMEM((CHUNK, D), jnp.float32),
            pltpu.VMEM((CHUNK, D), jnp.float32),
            pltpu.VMEM_SHARED((N_PAD, D), jnp.float32),
            pltpu.SemaphoreType.DMA,
            pltpu.SemaphoreType.DMA,
            pltpu.SemaphoreType.DMA,
            pltpu.SemaphoreType.DMA,
            pltpu.SemaphoreType.DMA,
        ],
    )
    return k(y0, y1, src0R, dst0R, src1R, dst1R, zerosD)


BLK = 2000


def _mm_body(x0_ref, x1_ref, x2_ref, egoW_ref, egob_ref, W0_ref, W1_ref,
             hidden_ref, xw0_ref, xw1_ref):
    hidden_ref[...] = (
        jnp.dot(x0_ref[...], egoW_ref[...], preferred_element_type=jnp.float32)
        + egob_ref[...]
    )
    xw0_ref[...] = jnp.dot(x1_ref[...], W0_ref[...],
                           preferred_element_type=jnp.float32)
    xw1_ref[...] = jnp.dot(x2_ref[...], W1_ref[...],
                           preferred_element_type=jnp.float32)


def _scale_body(xw0_ref, xw1_ref, d0_ref, d1_ref, y0_ref, y1_ref):
    y0_ref[...] = lax.rsqrt(d0_ref[:, 0:1] + 1.0) * xw0_ref[...]
    y1_ref[...] = lax.rsqrt(d1_ref[:, 0:1] + 1.0) * xw1_ref[...]


def _final_body(hidden_ref, y0_ref, y1_ref, a0_ref, a1_ref,
                d0_ref, d1_ref, b0_ref, b1_ref, out_ref):
    dinv0 = lax.rsqrt(d0_ref[:, 0:1] + 1.0)
    p0 = dinv0 * (a0_ref[...] + y0_ref[...]) + b0_ref[...]
    dinv1 = lax.rsqrt(d1_ref[:, 0:1] + 1.0)
    p1 = dinv1 * (a1_ref[...] + y1_ref[...]) + b1_ref[...]
    out_ref[...] = (hidden_ref[...] + jnp.maximum(p0, 0.0) + jnp.maximum(p1, 0.0))


def _row_spec(w):
    return pl.BlockSpec((BLK, w), lambda i: (i, 0))


def _full_spec(h, w):
    return pl.BlockSpec((h, w), lambda i: (0, 0))


def _mm_call(x0, x1, x2, egoW, egob, W0, W1):
    outs = [jax.ShapeDtypeStruct((N, D), jnp.float32)] * 3
    return pl.pallas_call(
        _mm_body,
        grid=(N // BLK,),
        in_specs=[
            _row_spec(D), _row_spec(D), _row_spec(D),
            _full_spec(D, D), _full_spec(1, D), _full_spec(D, D), _full_spec(D, D),
        ],
        out_specs=[_row_spec(D)] * 3,
        out_shape=outs,
    )(x0, x1, x2, egoW, egob, W0, W1)


def _scale_call(xw0, xw1, d0, d1):
    outs = [jax.ShapeDtypeStruct((N, D), jnp.float32)] * 2
    return pl.pallas_call(
        _scale_body,
        grid=(N // BLK,),
        in_specs=[_row_spec(D), _row_spec(D), _row_spec(D), _row_spec(D)],
        out_specs=[_row_spec(D)] * 2,
        out_shape=outs,
    )(xw0, xw1, d0, d1)


def _final_call(hidden, y0, y1, a0, a1, d0, d1, b0, b1):
    return pl.pallas_call(
        _final_body,
        grid=(N // BLK,),
        in_specs=[
            _row_spec(D), _row_spec(D), _row_spec(D),
            _row_spec(D), _row_spec(D),
            _row_spec(D), _row_spec(D),
            _full_spec(1, D), _full_spec(1, D),
        ],
        out_specs=_row_spec(D),
        out_shape=jax.ShapeDtypeStruct((N, D), jnp.float32),
    )(hidden, y0, y1, a0, a1, d0, d1, b0, b1)


def kernel(x0, x1, x2, ei0, ei1, ego_W, ego_b, W0, b0, W1, b1):
    src0R = ei0[0].reshape(E // CHUNK, CHUNK)
    dst0R = ei0[1].reshape(E // CHUNK, CHUNK)
    src1R = ei1[0].reshape(E // CHUNK, CHUNK)
    dst1R = ei1[1].reshape(E // CHUNK, CHUNK)
    onesD = jnp.ones((CHUNK, D), jnp.float32)
    zerosD = jnp.zeros((R_SUB, D), jnp.float32)

    deg0, deg1 = _deg_call(dst0R, dst1R, onesD, zerosD)
    d0, d1 = deg0[:N], deg1[:N]

    hidden, xw0, xw1 = _mm_call(x0, x1, x2, ego_W, ego_b.reshape(1, D), W0, W1)
    y0, y1 = _scale_call(xw0, xw1, d0, d1)

    acc0, acc1 = _pay_call(y0, y1, src0R, dst0R, src1R, dst1R, zerosD)
    a0, a1 = acc0[:N], acc1[:N]

    return _final_call(hidden, y0, y1, a0, a1, d0, d1,
                       b0.reshape(1, D), b1.reshape(1, D))


# speedup vs baseline: 31.5517x; 1.0694x over previous
import functools

import jax
import jax.numpy as jnp
from jax import lax
from jax.experimental import pallas as pl
from jax.experimental.pallas import tpu as pltpu
from jax.experimental.pallas import tpu_sc as plsc

N = 10000
D = 128
E = 320000
NC = 2
NS = 16
NW = NC * NS
N_PAD = 10240
R_SUB = N_PAD // NS
CHUNK = 100
E_PER_S = E // NS
NCH = E_PER_S // CHUNK
MAIN = (NCH // 4) * 4 - 4

_mesh = plsc.VectorSubcoreMesh(core_axis_name="c", subcore_axis_name="s")


DEG_GRP = 10


def _deg_body(e0R_hbm, e1R_hbm, ones_hbm, zeros_hbm, out0_hbm, out1_hbm,
              idx_v, ones_v, sh, sem):
    c = lax.axis_index("c")
    s = lax.axis_index("s")
    pltpu.sync_copy(zeros_hbm, sh.at[pl.ds(s * R_SUB, R_SUB)])
    pltpu.sync_copy(ones_hbm, ones_v)

    def count(eR_hbm):
        pltpu.sync_copy(eR_hbm.at[1, pl.ds(s * NCH, NCH)], idx_v)
        plsc.subcore_barrier()

        @pl.loop(0, NCH // DEG_GRP)
        def _(g):
            for j in range(DEG_GRP):
                pltpu.async_copy(ones_v, sh.at[idx_v.at[g * DEG_GRP + j]],
                                 sem, add=True)
            for j in range(DEG_GRP):
                pltpu.make_async_copy(ones_v, sh.at[idx_v.at[g * DEG_GRP + j]],
                                      sem).wait()

    @pl.when(c == 0)
    def _():
        count(e0R_hbm)

    @pl.when(c == 1)
    def _():
        count(e1R_hbm)

    plsc.subcore_barrier()
    sl = pl.ds(s * R_SUB, R_SUB)

    @pl.when(c == 0)
    def _():
        pltpu.sync_copy(sh.at[sl], out0_hbm.at[sl])

    @pl.when(c == 1)
    def _():
        pltpu.sync_copy(sh.at[sl], out1_hbm.at[sl])


def _deg_call(e0R, e1R, onesD, zerosD):
    out = jax.ShapeDtypeStruct((N_PAD, D), jnp.float32)
    k = pl.kernel(
        _deg_body,
        out_type=(out, out),
        mesh=_mesh,
        scratch_types=[
            pltpu.VMEM((NCH, CHUNK), jnp.int32),
            pltpu.VMEM((CHUNK, D), jnp.float32),
            pltpu.VMEM_SHARED((N_PAD, D), jnp.float32),
            pltpu.SemaphoreType.DMA,
        ],
    )
    return k(e0R, e1R, onesD, zerosD)


SG = 8
NSTAGE = NCH // SG
NPAIR = NCH // 2


def _pay_body(y0_hbm, y1_hbm, e0R_hbm, e1R_hbm, zeros_hbm,
              out0_hbm, out1_hbm,
              srcA, dstA, srcB, dstB, rows0, rows1, sh,
              gsem0, gsem1, ssem0, ssem1, isem):
    c = lax.axis_index("c")
    s = lax.axis_index("s")

    def pipeline(y_hbm, eR_hbm, out_hbm):
        base = s * NCH
        pltpu.sync_copy(zeros_hbm, sh.at[pl.ds(s * R_SUB, R_SUB)])
        pltpu.sync_copy(eR_hbm.at[0, pl.ds(base, SG)], srcA)
        pltpu.sync_copy(eR_hbm.at[1, pl.ds(base, SG)], dstA)
        pltpu.async_copy(eR_hbm.at[0, pl.ds(base + SG, SG)], srcB, isem)
        pltpu.async_copy(eR_hbm.at[1, pl.ds(base + SG, SG)], dstB, isem)
        plsc.subcore_barrier()

        def g_start(srcI, r, rows, gsem):
            pltpu.async_copy(y_hbm.at[srcI.at[r]], rows, gsem)

        def g_wait(srcI, r, rows, gsem):
            pltpu.make_async_copy(y_hbm.at[srcI.at[r]], rows, gsem).wait()

        def s_fire(dstI, r, rows, ssem):
            pltpu.async_copy(rows, sh.at[dstI.at[r]], ssem, add=True)

        def s_drain(dstI, r, rows, ssem):
            pltpu.make_async_copy(rows, sh.at[dstI.at[r]], ssem).wait()

        def i_start(srcI, dstI, t):
            pltpu.async_copy(eR_hbm.at[0, pl.ds(base + t * SG, SG)], srcI, isem)
            pltpu.async_copy(eR_hbm.at[1, pl.ds(base + t * SG, SG)], dstI, isem)

        def i_wait(srcI, dstI, t):
            pltpu.make_async_copy(eR_hbm.at[0, pl.ds(base + t * SG, SG)],
                                  srcI, isem).wait()
            pltpu.make_async_copy(eR_hbm.at[1, pl.ds(base + t * SG, SG)],
                                  dstI, isem).wait()

        g_start(srcA, 0, rows0, gsem0)

        def pair_body(p, srcC, dstC, srcN, dstN):
            q = p % 4
            r0 = q * 2
            r1 = q * 2 + 1
            t = p // 4

            @pl.when(jnp.logical_and(p >= 1, q >= 1))
            def _():
                s_drain(dstC, r1 - 2, rows1, ssem1)

            @pl.when(jnp.logical_and(p >= 1, q == 0))
            def _():
                s_drain(dstN, SG - 1, rows1, ssem1)

            @pl.when(jnp.logical_and(q == 0, t + 1 <= NSTAGE - 1))
            def _():
                @pl.when(p >= 4)
                def _():
                    i_start(srcN, dstN, t + 1)

            g_start(srcC, r1, rows1, gsem1)
            g_wait(srcC, r0, rows0, gsem0)
            s_fire(dstC, r0, rows0, ssem0)
            s_drain(dstC, r0, rows0, ssem0)

            @pl.when(jnp.logical_and(q < 3, p < NPAIR - 1))
            def _():
                g_start(srcC, r0 + 2, rows0, gsem0)

            @pl.when(jnp.logical_and(q == 3, p < NPAIR - 1))
            def _():
                i_wait(srcN, dstN, t + 1)
                g_start(srcN, 0, rows0, gsem0)

            g_wait(srcC, r1, rows1, gsem1)
            s_fire(dstC, r1, rows1, ssem1)

        @pl.loop(0, NPAIR)
        def _(p):
            slot = (p // 4) % 2

            @pl.when(slot == 0)
            def _():
                pair_body(p, srcA, dstA, srcB, dstB)

            @pl.when(slot == 1)
            def _():
                pair_body(p, srcB, dstB, srcA, dstA)

        s_drain(dstA, SG - 1, rows1, ssem1)

        plsc.subcore_barrier()
        sl = pl.ds(s * R_SUB, R_SUB)
        pltpu.sync_copy(sh.at[sl], out_hbm.at[sl])

    @pl.when(c == 0)
    def _():
        pipeline(y0_hbm, e0R_hbm, out0_hbm)

    @pl.when(c == 1)
    def _():
        pipeline(y1_hbm, e1R_hbm, out1_hbm)


def _pay_call(y0, y1, e0R, e1R, zerosD):
    out = jax.ShapeDtypeStruct((N_PAD, D), jnp.float32)
    k = pl.kernel(
        _pay_body,
        out_type=(out, out),
        mesh=_mesh,
        scratch_types=[
            pltpu.VMEM((SG, CHUNK), jnp.int32),
            pltpu.VMEM((SG, CHUNK), jnp.int32),
            pltpu.VMEM((SG, CHUNK), jnp.int32),
            pltpu.VMEM((SG, CHUNK), jnp.int32),
            pltpu.VMEM((CHUNK, D), jnp.float32),
            pltpu.VMEM((CHUNK, D), jnp.float32),
            pltpu.VMEM_SHARED((N_PAD, D), jnp.float32),
            pltpu.SemaphoreType.DMA,
            pltpu.SemaphoreType.DMA,
            pltpu.SemaphoreType.DMA,
            pltpu.SemaphoreType.DMA,
            pltpu.SemaphoreType.DMA,
        ],
    )
    return k(y0, y1, e0R, e1R, zerosD)


BLK = 2000


def _mm_body(x0_ref, x1_ref, x2_ref, egoW_ref, egob_ref, W0_ref, W1_ref,
             hidden_ref, xw0_ref, xw1_ref):
    hidden_ref[...] = (
        jnp.dot(x0_ref[...], egoW_ref[...], preferred_element_type=jnp.float32)
        + egob_ref[...]
    )
    xw0_ref[...] = jnp.dot(x1_ref[...], W0_ref[...],
                           preferred_element_type=jnp.float32)
    xw1_ref[...] = jnp.dot(x2_ref[...], W1_ref[...],
                           preferred_element_type=jnp.float32)


def _scale_body(xw0_ref, xw1_ref, d0_ref, d1_ref, y0_ref, y1_ref):
    y0_ref[...] = lax.rsqrt(d0_ref[:, 0:1] + 1.0) * xw0_ref[...]
    y1_ref[...] = lax.rsqrt(d1_ref[:, 0:1] + 1.0) * xw1_ref[...]


def _final_body(hidden_ref, y0_ref, y1_ref, a0_ref, a1_ref,
                d0_ref, d1_ref, b0_ref, b1_ref, out_ref):
    dinv0 = lax.rsqrt(d0_ref[:, 0:1] + 1.0)
    p0 = dinv0 * (a0_ref[...] + y0_ref[...]) + b0_ref[...]
    dinv1 = lax.rsqrt(d1_ref[:, 0:1] + 1.0)
    p1 = dinv1 * (a1_ref[...] + y1_ref[...]) + b1_ref[...]
    out_ref[...] = (hidden_ref[...] + jnp.maximum(p0, 0.0) + jnp.maximum(p1, 0.0))


def _row_spec(w):
    return pl.BlockSpec((BLK, w), lambda i: (i, 0))


def _full_spec(h, w):
    return pl.BlockSpec((h, w), lambda i: (0, 0))


def _mm_call(x0, x1, x2, egoW, egob, W0, W1):
    outs = [jax.ShapeDtypeStruct((N, D), jnp.float32)] * 3
    return pl.pallas_call(
        _mm_body,
        grid=(N // BLK,),
        in_specs=[
            _row_spec(D), _row_spec(D), _row_spec(D),
            _full_spec(D, D), _full_spec(1, D), _full_spec(D, D), _full_spec(D, D),
        ],
        out_specs=[_row_spec(D)] * 3,
        out_shape=outs,
    )(x0, x1, x2, egoW, egob, W0, W1)


def _scale_call(xw0, xw1, d0, d1):
    outs = [jax.ShapeDtypeStruct((N, D), jnp.float32)] * 2
    return pl.pallas_call(
        _scale_body,
        grid=(N // BLK,),
        in_specs=[_row_spec(D), _row_spec(D), _row_spec(D), _row_spec(D)],
        out_specs=[_row_spec(D)] * 2,
        out_shape=outs,
    )(xw0, xw1, d0, d1)


def _final_call(hidden, y0, y1, a0, a1, d0, d1, b0, b1):
    return pl.pallas_call(
        _final_body,
        grid=(N // BLK,),
        in_specs=[
            _row_spec(D), _row_spec(D), _row_spec(D),
            _row_spec(D), _row_spec(D),
            _row_spec(D), _row_spec(D),
            _full_spec(1, D), _full_spec(1, D),
        ],
        out_specs=_row_spec(D),
        out_shape=jax.ShapeDtypeStruct((N, D), jnp.float32),
    )(hidden, y0, y1, a0, a1, d0, d1, b0, b1)


def kernel(x0, x1, x2, ei0, ei1, ego_W, ego_b, W0, b0, W1, b1):
    e0R = ei0.reshape(2, E // CHUNK, CHUNK)
    e1R = ei1.reshape(2, E // CHUNK, CHUNK)
    onesD = jnp.ones((CHUNK, D), jnp.float32)
    zerosD = jnp.zeros((R_SUB, D), jnp.float32)

    deg0, deg1 = _deg_call(e0R, e1R, onesD, zerosD)

    hidden, xw0, xw1 = _mm_call(x0, x1, x2, ego_W, ego_b.reshape(1, D), W0, W1)
    y0, y1 = _scale_call(xw0, xw1, deg0, deg1)

    acc0, acc1 = _pay_call(y0, y1, e0R, e1R, zerosD)

    return _final_call(hidden, y0, y1, acc0, acc1, deg0, deg1,
                       b0.reshape(1, D), b1.reshape(1, D))


# speedup vs baseline: 42.8537x; 1.3582x over previous
import functools

import jax
import jax.numpy as jnp
from jax import lax
from jax.experimental import pallas as pl
from jax.experimental.pallas import tpu as pltpu
from jax.experimental.pallas import tpu_sc as plsc

N = 10000
D = 128
E = 320000
NC = 2
NS = 16
NW = NC * NS
N_PAD = 10240
R_SUB = N_PAD // NS
CHUNK = 100
E_PER_S = E // NS
NCH = E_PER_S // CHUNK
MAIN = (NCH // 4) * 4 - 4

_mesh = plsc.VectorSubcoreMesh(core_axis_name="c", subcore_axis_name="s")


def _deg_body(ef0_hbm, ef1_hbm, out0_hbm, out1_hbm,
              idx_v, hist_v, red_v, red_sh, isem):
    c = lax.axis_index("c")
    s = lax.axis_index("s")

    def count(ef_hbm, out_hbm):
        src_slice = pl.ds(E + s * E_PER_S, E_PER_S)
        pltpu.async_copy(ef_hbm.at[src_slice], idx_v, isem)
        zeros16 = jnp.zeros((16,), jnp.float32)

        @pl.loop(0, N_PAD // 16)
        def _(i):
            hist_v[pl.ds(i * 16, 16)] = zeros16

        pltpu.make_async_copy(ef_hbm.at[src_slice], idx_v, isem).wait()
        ones16 = jnp.ones((16,), jnp.float32)

        @pl.loop(0, E_PER_S // 16)
        def _(i):
            iv = idx_v[pl.ds(i * 16, 16)]
            plsc.addupdate_scatter(hist_v, [iv], ones16)

        pltpu.sync_copy(hist_v, red_sh.at[pl.ds(s * N_PAD, N_PAD)])
        plsc.subcore_barrier()
        for r in range(NS):
            pltpu.async_copy(red_sh.at[pl.ds(r * N_PAD + s * R_SUB, R_SUB)],
                             red_v.at[pl.ds(r * R_SUB, R_SUB)], isem)
        for r in range(NS):
            pltpu.make_async_copy(red_sh.at[pl.ds(r * N_PAD + s * R_SUB, R_SUB)],
                                  red_v.at[pl.ds(r * R_SUB, R_SUB)], isem).wait()

        @pl.loop(0, R_SUB // 16)
        def _(l):
            v = red_v[pl.ds(l * 16, 16)]
            for r in range(1, NS):
                v = v + red_v[pl.ds(r * R_SUB + l * 16, 16)]
            hist_v[pl.ds(l * 16, 16)] = v

        pltpu.sync_copy(hist_v.at[pl.ds(0, R_SUB)],
                        out_hbm.at[pl.ds(s * R_SUB, R_SUB)])

    @pl.when(c == 0)
    def _():
        count(ef0_hbm, out0_hbm)

    @pl.when(c == 1)
    def _():
        count(ef1_hbm, out1_hbm)


def _deg_call(ef0, ef1):
    import dataclasses
    cp = pltpu.CompilerParams()
    if "needs_layout_passes" in pltpu.CompilerParams.__dataclass_fields__:
        cp = dataclasses.replace(cp, needs_layout_passes=False)
    out = jax.ShapeDtypeStruct((N_PAD,), jnp.float32)
    k = pl.kernel(
        _deg_body,
        out_type=(out, out),
        mesh=_mesh,
        scratch_types=[
            pltpu.VMEM((E_PER_S,), jnp.int32),
            pltpu.VMEM((N_PAD,), jnp.float32),
            pltpu.VMEM((NS * R_SUB,), jnp.float32),
            pltpu.VMEM_SHARED((NS * N_PAD,), jnp.float32),
            pltpu.SemaphoreType.DMA,
        ],
        compiler_params=cp,
    )
    return k(ef0, ef1)


SG = 8
NSTAGE = NCH // SG
NPAIR = NCH // 2


def _pay_body(y0_hbm, y1_hbm, e0R_hbm, e1R_hbm, zeros_hbm,
              out0_hbm, out1_hbm,
              srcA, dstA, srcB, dstB, rows0, rows1, sh,
              gsem0, gsem1, ssem0, ssem1, isem):
    c = lax.axis_index("c")
    s = lax.axis_index("s")

    def pipeline(y_hbm, eR_hbm, out_hbm):
        base = s * NCH
        pltpu.sync_copy(zeros_hbm, sh.at[pl.ds(s * R_SUB, R_SUB)])
        pltpu.sync_copy(eR_hbm.at[0, pl.ds(base, SG)], srcA)
        pltpu.sync_copy(eR_hbm.at[1, pl.ds(base, SG)], dstA)
        pltpu.async_copy(eR_hbm.at[0, pl.ds(base + SG, SG)], srcB, isem)
        pltpu.async_copy(eR_hbm.at[1, pl.ds(base + SG, SG)], dstB, isem)
        plsc.subcore_barrier()

        def g_start(srcI, r, rows, gsem):
            pltpu.async_copy(y_hbm.at[srcI.at[r]], rows, gsem)

        def g_wait(srcI, r, rows, gsem):
            pltpu.make_async_copy(y_hbm.at[srcI.at[r]], rows, gsem).wait()

        def s_fire(dstI, r, rows, ssem):
            pltpu.async_copy(rows, sh.at[dstI.at[r]], ssem, add=True)

        def s_drain(dstI, r, rows, ssem):
            pltpu.make_async_copy(rows, sh.at[dstI.at[r]], ssem).wait()

        def i_start(srcI, dstI, t):
            pltpu.async_copy(eR_hbm.at[0, pl.ds(base + t * SG, SG)], srcI, isem)
            pltpu.async_copy(eR_hbm.at[1, pl.ds(base + t * SG, SG)], dstI, isem)

        def i_wait(srcI, dstI, t):
            pltpu.make_async_copy(eR_hbm.at[0, pl.ds(base + t * SG, SG)],
                                  srcI, isem).wait()
            pltpu.make_async_copy(eR_hbm.at[1, pl.ds(base + t * SG, SG)],
                                  dstI, isem).wait()

        g_start(srcA, 0, rows0, gsem0)

        def pair_body(p, srcC, dstC, srcN, dstN):
            q = p % 4
            r0 = q * 2
            r1 = q * 2 + 1
            t = p // 4

            @pl.when(jnp.logical_and(p >= 1, q >= 1))
            def _():
                s_drain(dstC, r1 - 2, rows1, ssem1)

            @pl.when(jnp.logical_and(p >= 1, q == 0))
            def _():
                s_drain(dstN, SG - 1, rows1, ssem1)

            @pl.when(jnp.logical_and(q == 0, t + 1 <= NSTAGE - 1))
            def _():
                @pl.when(p >= 4)
                def _():
                    i_start(srcN, dstN, t + 1)

            g_start(srcC, r1, rows1, gsem1)
            g_wait(srcC, r0, rows0, gsem0)
            s_fire(dstC, r0, rows0, ssem0)
            s_drain(dstC, r0, rows0, ssem0)

            @pl.when(jnp.logical_and(q < 3, p < NPAIR - 1))
            def _():
                g_start(srcC, r0 + 2, rows0, gsem0)

            @pl.when(jnp.logical_and(q == 3, p < NPAIR - 1))
            def _():
                i_wait(srcN, dstN, t + 1)
                g_start(srcN, 0, rows0, gsem0)

            g_wait(srcC, r1, rows1, gsem1)
            s_fire(dstC, r1, rows1, ssem1)

        @pl.loop(0, NPAIR)
        def _(p):
            slot = (p // 4) % 2

            @pl.when(slot == 0)
            def _():
                pair_body(p, srcA, dstA, srcB, dstB)

            @pl.when(slot == 1)
            def _():
                pair_body(p, srcB, dstB, srcA, dstA)

        s_drain(dstA, SG - 1, rows1, ssem1)

        plsc.subcore_barrier()
        sl = pl.ds(s * R_SUB, R_SUB)
        pltpu.sync_copy(sh.at[sl], out_hbm.at[sl])

    @pl.when(c == 0)
    def _():
        pipeline(y0_hbm, e0R_hbm, out0_hbm)

    @pl.when(c == 1)
    def _():
        pipeline(y1_hbm, e1R_hbm, out1_hbm)


def _pay_call(y0, y1, e0R, e1R, zerosD):
    out = jax.ShapeDtypeStruct((N_PAD, D), jnp.float32)
    k = pl.kernel(
        _pay_body,
        out_type=(out, out),
        mesh=_mesh,
        scratch_types=[
            pltpu.VMEM((SG, CHUNK), jnp.int32),
            pltpu.VMEM((SG, CHUNK), jnp.int32),
            pltpu.VMEM((SG, CHUNK), jnp.int32),
            pltpu.VMEM((SG, CHUNK), jnp.int32),
            pltpu.VMEM((CHUNK, D), jnp.float32),
            pltpu.VMEM((CHUNK, D), jnp.float32),
            pltpu.VMEM_SHARED((N_PAD, D), jnp.float32),
            pltpu.SemaphoreType.DMA,
            pltpu.SemaphoreType.DMA,
            pltpu.SemaphoreType.DMA,
            pltpu.SemaphoreType.DMA,
            pltpu.SemaphoreType.DMA,
        ],
    )
    return k(y0, y1, e0R, e1R, zerosD)


BLK = 2000


def _mm_body(x0_ref, x1_ref, x2_ref, egoW_ref, egob_ref, W0_ref, W1_ref,
             hidden_ref, xw0_ref, xw1_ref):
    hidden_ref[...] = (
        jnp.dot(x0_ref[...], egoW_ref[...], preferred_element_type=jnp.float32)
        + egob_ref[...]
    )
    xw0_ref[...] = jnp.dot(x1_ref[...], W0_ref[...],
                           preferred_element_type=jnp.float32)
    xw1_ref[...] = jnp.dot(x2_ref[...], W1_ref[...],
                           preferred_element_type=jnp.float32)


def _scale_body(xw0_ref, xw1_ref, d0_ref, d1_ref, y0_ref, y1_ref):
    y0_ref[...] = lax.rsqrt(d0_ref[...] + 1.0) * xw0_ref[...]
    y1_ref[...] = lax.rsqrt(d1_ref[...] + 1.0) * xw1_ref[...]


def _final_body(hidden_ref, y0_ref, y1_ref, a0_ref, a1_ref,
                d0_ref, d1_ref, b0_ref, b1_ref, out_ref):
    dinv0 = lax.rsqrt(d0_ref[...] + 1.0)
    p0 = dinv0 * (a0_ref[...] + y0_ref[...]) + b0_ref[...]
    dinv1 = lax.rsqrt(d1_ref[...] + 1.0)
    p1 = dinv1 * (a1_ref[...] + y1_ref[...]) + b1_ref[...]
    out_ref[...] = (hidden_ref[...] + jnp.maximum(p0, 0.0) + jnp.maximum(p1, 0.0))


def _row_spec(w):
    return pl.BlockSpec((BLK, w), lambda i: (i, 0))


def _deg_spec():
    return pl.BlockSpec((BLK, 1), lambda i: (i, 0))


def _full_spec(h, w):
    return pl.BlockSpec((h, w), lambda i: (0, 0))


def _mm_call(x0, x1, x2, egoW, egob, W0, W1):
    outs = [jax.ShapeDtypeStruct((N, D), jnp.float32)] * 3
    return pl.pallas_call(
        _mm_body,
        grid=(N // BLK,),
        in_specs=[
            _row_spec(D), _row_spec(D), _row_spec(D),
            _full_spec(D, D), _full_spec(1, D), _full_spec(D, D), _full_spec(D, D),
        ],
        out_specs=[_row_spec(D)] * 3,
        out_shape=outs,
    )(x0, x1, x2, egoW, egob, W0, W1)


def _scale_call(xw0, xw1, d0, d1):
    outs = [jax.ShapeDtypeStruct((N, D), jnp.float32)] * 2
    return pl.pallas_call(
        _scale_body,
        grid=(N // BLK,),
        in_specs=[_row_spec(D), _row_spec(D), _deg_spec(), _deg_spec()],
        out_specs=[_row_spec(D)] * 2,
        out_shape=outs,
    )(xw0, xw1, d0, d1)


def _final_call(hidden, y0, y1, a0, a1, d0, d1, b0, b1):
    return pl.pallas_call(
        _final_body,
        grid=(N // BLK,),
        in_specs=[
            _row_spec(D), _row_spec(D), _row_spec(D),
            _row_spec(D), _row_spec(D),
            _deg_spec(), _deg_spec(),
            _full_spec(1, D), _full_spec(1, D),
        ],
        out_specs=_row_spec(D),
        out_shape=jax.ShapeDtypeStruct((N, D), jnp.float32),
    )(hidden, y0, y1, a0, a1, d0, d1, b0, b1)


def kernel(x0, x1, x2, ei0, ei1, ego_W, ego_b, W0, b0, W1, b1):
    e0R = ei0.reshape(2, E // CHUNK, CHUNK)
    e1R = ei1.reshape(2, E // CHUNK, CHUNK)
    onesD = jnp.ones((CHUNK, D), jnp.float32)
    zerosD = jnp.zeros((R_SUB, D), jnp.float32)

    deg0, deg1 = _deg_call(ei0.reshape(2 * E), ei1.reshape(2 * E))
    deg0 = deg0.reshape(N_PAD, 1)
    deg1 = deg1.reshape(N_PAD, 1)

    hidden, xw0, xw1 = _mm_call(x0, x1, x2, ego_W, ego_b.reshape(1, D), W0, W1)
    y0, y1 = _scale_call(xw0, xw1, deg0, deg1)

    acc0, acc1 = _pay_call(y0, y1, e0R, e1R, zerosD)

    return _final_call(hidden, y0, y1, acc0, acc1, deg0, deg1,
                       b0.reshape(1, D), b1.reshape(1, D))


# speedup vs baseline: 44.4474x; 1.0372x over previous
import functools

import jax
import jax.numpy as jnp
from jax import lax
from jax.experimental import pallas as pl
from jax.experimental.pallas import tpu as pltpu
from jax.experimental.pallas import tpu_sc as plsc

N = 10000
D = 128
E = 320000
NC = 2
NS = 16
NW = NC * NS
N_PAD = 10240
R_SUB = N_PAD // NS
CHUNK = 100
E_PER_S = E // NS
NCH = E_PER_S // CHUNK
MAIN = (NCH // 4) * 4 - 4

_mesh = plsc.VectorSubcoreMesh(core_axis_name="c", subcore_axis_name="s")


def _deg_body(ef0_hbm, ef1_hbm, out0_hbm, out1_hbm,
              idx_v, hist_v, red_v, red_sh, isem):
    c = lax.axis_index("c")
    s = lax.axis_index("s")

    def count(ef_hbm, out_hbm):
        src_slice = pl.ds(E + s * E_PER_S, E_PER_S)
        pltpu.async_copy(ef_hbm.at[src_slice], idx_v, isem)
        zeros16 = jnp.zeros((16,), jnp.float32)

        @pl.loop(0, N_PAD // 16)
        def _(i):
            hist_v[pl.ds(i * 16, 16)] = zeros16

        pltpu.make_async_copy(ef_hbm.at[src_slice], idx_v, isem).wait()
        ones16 = jnp.ones((16,), jnp.float32)

        @pl.loop(0, E_PER_S // 16)
        def _(i):
            iv = idx_v[pl.ds(i * 16, 16)]
            plsc.addupdate_scatter(hist_v, [iv], ones16)

        pltpu.sync_copy(hist_v, red_sh.at[pl.ds(s * N_PAD, N_PAD)])
        plsc.subcore_barrier()
        for r in range(NS):
            pltpu.async_copy(red_sh.at[pl.ds(r * N_PAD + s * R_SUB, R_SUB)],
                             red_v.at[pl.ds(r * R_SUB, R_SUB)], isem)
        for r in range(NS):
            pltpu.make_async_copy(red_sh.at[pl.ds(r * N_PAD + s * R_SUB, R_SUB)],
                                  red_v.at[pl.ds(r * R_SUB, R_SUB)], isem).wait()

        @pl.loop(0, R_SUB // 16)
        def _(l):
            v = red_v[pl.ds(l * 16, 16)]
            for r in range(1, NS):
                v = v + red_v[pl.ds(r * R_SUB + l * 16, 16)]
            hist_v[pl.ds(l * 16, 16)] = v

        pltpu.sync_copy(hist_v.at[pl.ds(0, R_SUB)],
                        out_hbm.at[pl.ds(s * R_SUB, R_SUB)])

    @pl.when(c == 0)
    def _():
        count(ef0_hbm, out0_hbm)

    @pl.when(c == 1)
    def _():
        count(ef1_hbm, out1_hbm)


def _deg_call(ef0, ef1):
    import dataclasses
    cp = pltpu.CompilerParams()
    if "needs_layout_passes" in pltpu.CompilerParams.__dataclass_fields__:
        cp = dataclasses.replace(cp, needs_layout_passes=False)
    out = jax.ShapeDtypeStruct((N_PAD,), jnp.float32)
    k = pl.kernel(
        _deg_body,
        out_type=(out, out),
        mesh=_mesh,
        scratch_types=[
            pltpu.VMEM((E_PER_S,), jnp.int32),
            pltpu.VMEM((N_PAD,), jnp.float32),
            pltpu.VMEM((NS * R_SUB,), jnp.float32),
            pltpu.VMEM_SHARED((NS * N_PAD,), jnp.float32),
            pltpu.SemaphoreType.DMA,
        ],
        compiler_params=cp,
    )
    return k(ef0, ef1)


SG = 8
NSTAGE = NCH // SG
NPAIR = NCH // 2


def _pay_body(y0_hbm, y1_hbm, e0R_hbm, e1R_hbm, zeros_hbm,
              out0_hbm, out1_hbm,
              srcA, dstA, srcB, dstB, rows0, rows1, sh,
              gsem0, gsem1, ssem0, ssem1, isem):
    c = lax.axis_index("c")
    s = lax.axis_index("s")

    def pipeline(y_hbm, eR_hbm, out_hbm):
        base = s * NCH
        pltpu.sync_copy(zeros_hbm, sh.at[pl.ds(s * R_SUB, R_SUB)])
        pltpu.sync_copy(eR_hbm.at[0, pl.ds(base, SG)], srcA)
        pltpu.sync_copy(eR_hbm.at[1, pl.ds(base, SG)], dstA)
        pltpu.async_copy(eR_hbm.at[0, pl.ds(base + SG, SG)], srcB, isem)
        pltpu.async_copy(eR_hbm.at[1, pl.ds(base + SG, SG)], dstB, isem)
        plsc.subcore_barrier()

        def g_start(srcI, r, rows, gsem):
            pltpu.async_copy(y_hbm.at[srcI.at[r]], rows, gsem)

        def g_wait(srcI, r, rows, gsem):
            pltpu.make_async_copy(y_hbm.at[srcI.at[r]], rows, gsem).wait()

        def s_fire(dstI, r, rows, ssem):
            pltpu.async_copy(rows, sh.at[dstI.at[r]], ssem, add=True)

        def s_drain(dstI, r, rows, ssem):
            pltpu.make_async_copy(rows, sh.at[dstI.at[r]], ssem).wait()

        def i_start(srcI, dstI, t):
            pltpu.async_copy(eR_hbm.at[0, pl.ds(base + t * SG, SG)], srcI, isem)
            pltpu.async_copy(eR_hbm.at[1, pl.ds(base + t * SG, SG)], dstI, isem)

        def i_wait(srcI, dstI, t):
            pltpu.make_async_copy(eR_hbm.at[0, pl.ds(base + t * SG, SG)],
                                  srcI, isem).wait()
            pltpu.make_async_copy(eR_hbm.at[1, pl.ds(base + t * SG, SG)],
                                  dstI, isem).wait()

        g_start(srcA, 0, rows0, gsem0)

        def pair_body(p, srcC, dstC, srcN, dstN):
            q = p % 4
            r0 = q * 2
            r1 = q * 2 + 1
            t = p // 4

            @pl.when(jnp.logical_and(p >= 1, q >= 1))
            def _():
                s_drain(dstC, r1 - 2, rows1, ssem1)

            @pl.when(jnp.logical_and(p >= 1, q == 0))
            def _():
                s_drain(dstN, SG - 1, rows1, ssem1)

            @pl.when(jnp.logical_and(q == 0, t + 1 <= NSTAGE - 1))
            def _():
                @pl.when(p >= 4)
                def _():
                    i_start(srcN, dstN, t + 1)

            g_start(srcC, r1, rows1, gsem1)
            g_wait(srcC, r0, rows0, gsem0)
            s_fire(dstC, r0, rows0, ssem0)
            s_drain(dstC, r0, rows0, ssem0)

            @pl.when(jnp.logical_and(q < 3, p < NPAIR - 1))
            def _():
                g_start(srcC, r0 + 2, rows0, gsem0)

            @pl.when(jnp.logical_and(q == 3, p < NPAIR - 1))
            def _():
                i_wait(srcN, dstN, t + 1)
                g_start(srcN, 0, rows0, gsem0)

            g_wait(srcC, r1, rows1, gsem1)
            s_fire(dstC, r1, rows1, ssem1)

        @pl.loop(0, NPAIR)
        def _(p):
            slot = (p // 4) % 2

            @pl.when(slot == 0)
            def _():
                pair_body(p, srcA, dstA, srcB, dstB)

            @pl.when(slot == 1)
            def _():
                pair_body(p, srcB, dstB, srcA, dstA)

        s_drain(dstA, SG - 1, rows1, ssem1)

        plsc.subcore_barrier()
        sl = pl.ds(s * R_SUB, R_SUB)
        pltpu.sync_copy(sh.at[sl], out_hbm.at[sl])

    @pl.when(c == 0)
    def _():
        pipeline(y0_hbm, e0R_hbm, out0_hbm)

    @pl.when(c == 1)
    def _():
        pipeline(y1_hbm, e1R_hbm, out1_hbm)


def _pay_call(y0, y1, e0R, e1R, zerosD):
    out = jax.ShapeDtypeStruct((N_PAD, D), jnp.float32)
    k = pl.kernel(
        _pay_body,
        out_type=(out, out),
        mesh=_mesh,
        scratch_types=[
            pltpu.VMEM((SG, CHUNK), jnp.int32),
            pltpu.VMEM((SG, CHUNK), jnp.int32),
            pltpu.VMEM((SG, CHUNK), jnp.int32),
            pltpu.VMEM((SG, CHUNK), jnp.int32),
            pltpu.VMEM((CHUNK, D), jnp.float32),
            pltpu.VMEM((CHUNK, D), jnp.float32),
            pltpu.VMEM_SHARED((N_PAD, D), jnp.float32),
            pltpu.SemaphoreType.DMA,
            pltpu.SemaphoreType.DMA,
            pltpu.SemaphoreType.DMA,
            pltpu.SemaphoreType.DMA,
            pltpu.SemaphoreType.DMA,
        ],
    )
    return k(y0, y1, e0R, e1R, zerosD)


BLK = 2000


def _prep_body(x0_ref, x1_ref, x2_ref, egoW_ref, egob_ref, W0_ref, W1_ref,
               d0_ref, d1_ref, hidden_ref, y0_ref, y1_ref):
    hidden_ref[...] = (
        jnp.dot(x0_ref[...], egoW_ref[...], preferred_element_type=jnp.float32)
        + egob_ref[...]
    )
    y0_ref[...] = lax.rsqrt(d0_ref[...] + 1.0) * jnp.dot(
        x1_ref[...], W0_ref[...], preferred_element_type=jnp.float32)
    y1_ref[...] = lax.rsqrt(d1_ref[...] + 1.0) * jnp.dot(
        x2_ref[...], W1_ref[...], preferred_element_type=jnp.float32)


def _final_body(hidden_ref, y0_ref, y1_ref, a0_ref, a1_ref,
                d0_ref, d1_ref, b0_ref, b1_ref, out_ref):
    dinv0 = lax.rsqrt(d0_ref[...] + 1.0)
    p0 = dinv0 * (a0_ref[...] + y0_ref[...]) + b0_ref[...]
    dinv1 = lax.rsqrt(d1_ref[...] + 1.0)
    p1 = dinv1 * (a1_ref[...] + y1_ref[...]) + b1_ref[...]
    out_ref[...] = (hidden_ref[...] + jnp.maximum(p0, 0.0) + jnp.maximum(p1, 0.0))


def _row_spec(w):
    return pl.BlockSpec((BLK, w), lambda i: (i, 0))


def _deg_spec():
    return pl.BlockSpec((BLK, 1), lambda i: (i, 0))


def _full_spec(h, w):
    return pl.BlockSpec((h, w), lambda i: (0, 0))


def _prep_call(x0, x1, x2, egoW, egob, W0, W1, d0, d1):
    outs = [jax.ShapeDtypeStruct((N, D), jnp.float32)] * 3
    return pl.pallas_call(
        _prep_body,
        grid=(N // BLK,),
        in_specs=[
            _row_spec(D), _row_spec(D), _row_spec(D),
            _full_spec(D, D), _full_spec(1, D), _full_spec(D, D), _full_spec(D, D),
            _deg_spec(), _deg_spec(),
        ],
        out_specs=[_row_spec(D)] * 3,
        out_shape=outs,
    )(x0, x1, x2, egoW, egob, W0, W1, d0, d1)


def _final_call(hidden, y0, y1, a0, a1, d0, d1, b0, b1):
    return pl.pallas_call(
        _final_body,
        grid=(N // BLK,),
        in_specs=[
            _row_spec(D), _row_spec(D), _row_spec(D),
            _row_spec(D), _row_spec(D),
            _deg_spec(), _deg_spec(),
            _full_spec(1, D), _full_spec(1, D),
        ],
        out_specs=_row_spec(D),
        out_shape=jax.ShapeDtypeStruct((N, D), jnp.float32),
    )(hidden, y0, y1, a0, a1, d0, d1, b0, b1)


def kernel(x0, x1, x2, ei0, ei1, ego_W, ego_b, W0, b0, W1, b1):
    e0R = ei0.reshape(2, E // CHUNK, CHUNK)
    e1R = ei1.reshape(2, E // CHUNK, CHUNK)
    onesD = jnp.ones((CHUNK, D), jnp.float32)
    zerosD = jnp.zeros((R_SUB, D), jnp.float32)

    deg0, deg1 = _deg_call(ei0.reshape(2 * E), ei1.reshape(2 * E))
    deg0 = deg0.reshape(N_PAD, 1)
    deg1 = deg1.reshape(N_PAD, 1)

    hidden, y0, y1 = _prep_call(x0, x1, x2, ego_W, ego_b.reshape(1, D), W0, W1,
                                deg0, deg1)

    acc0, acc1 = _pay_call(y0, y1, e0R, e1R, zerosD)

    return _final_call(hidden, y0, y1, acc0, acc1, deg0, deg1,
                       b0.reshape(1, D), b1.reshape(1, D))
